# Initial kernel scaffold; baseline (speedup 1.0000x reference)
import jax
import jax.numpy as jnp
from jax.experimental import pallas as pl


def kernel(adj_indices, adj_values, features, W1, b1, W2, b2):
    raise NotImplementedError("write your pallas kernel here")


# speedup vs baseline: 9.1698x; 9.1698x over previous
import functools

import jax
import jax.numpy as jnp
from jax import lax
from jax.experimental import pallas as pl
from jax.experimental.pallas import tpu as pltpu
from jax.experimental.pallas import tpu_sc as plsc

NC = 2
NS = 16
NW = NC * NS
LANES = 16
GROUP = 80

_BCAST_DNUMS = lax.GatherDimensionNumbers(
    offset_dims=(), collapsed_slice_dims=(0,), start_index_map=(0,))


def _lane_broadcast(v, l):
  idx = jnp.full((LANES, 1), l, jnp.int32)
  return lax.gather(v, idx, _BCAST_DNUMS, (1,),
                    mode=lax.GatherScatterMode.PROMISE_IN_BOUNDS)


def _sc_spmm(table, src2, dst2, vals2, n_pad):
  ng_total = src2.shape[0]
  ng = ng_total // NW
  rpt = n_pad // NS
  mesh = plsc.VectorSubcoreMesh(core_axis_name="c", subcore_axis_name="s")

  @functools.partial(
      pl.kernel,
      out_type=jax.ShapeDtypeStruct((NC * n_pad, LANES), jnp.float32),
      mesh=mesh,
      scratch_types=[
          pltpu.VMEM((ng, GROUP), jnp.int32),
          pltpu.VMEM((ng, GROUP), jnp.int32),
          pltpu.VMEM((ng, GROUP), jnp.float32),
          pltpu.VMEM((GROUP, LANES), jnp.float32),
          pltpu.VMEM((rpt, LANES), jnp.float32),
          pltpu.VMEM_SHARED((n_pad, LANES), jnp.float32),
          pltpu.SemaphoreType.DMA,
      ],
      compiler_params=pltpu.CompilerParams(use_tc_tiling_on_sc=False),
  )
  def spmm(table_h, src_h, dst_h, vals_h, out_h,
           src_v, dst_v, vals_v, rows_v, zero_v, acc, sem):
    c = lax.axis_index("c")
    s = lax.axis_index("s")
    wid = c * NS + s

    def zbody(i, carry):
      zero_v[i] = jnp.zeros((LANES,), jnp.float32)
      return carry
    lax.fori_loop(0, rpt, zbody, 0)
    pltpu.sync_copy(zero_v, acc.at[pl.ds(s * rpt, rpt)])
    plsc.subcore_barrier()

    pltpu.sync_copy(src_h.at[pl.ds(wid * ng, ng)], src_v)
    pltpu.sync_copy(dst_h.at[pl.ds(wid * ng, ng)], dst_v)
    pltpu.sync_copy(vals_h.at[pl.ds(wid * ng, ng)], vals_v)

    def ebody(g, carry):
      pltpu.async_copy(table_h.at[src_v.at[g]], rows_v, sem).wait()
      for j in range(GROUP // LANES):
        v = vals_v[g, pl.ds(j * LANES, LANES)]
        for l in range(LANES):
          e = j * LANES + l
          rows_v[e] = rows_v[e] * _lane_broadcast(v, l)
      pltpu.sync_copy(rows_v, acc.at[dst_v.at[g]], add=True)
      return carry
    lax.fori_loop(0, ng, ebody, 0)

    plsc.subcore_barrier()
    pltpu.sync_copy(acc.at[pl.ds(s * rpt, rpt)],
                    out_h.at[pl.ds(c * n_pad + s * rpt, rpt)])

  return spmm(table, src2, dst2, vals2)


def _tc_linear1(x, w):
  m, k = x.shape
  n = w.shape[1]
  bm = 1000

  def body(x_ref, w_ref, o_ref):
    o_ref[...] = jnp.dot(x_ref[...], w_ref[...],
                         preferred_element_type=jnp.float32)

  return pl.pallas_call(
      body,
      grid=(m // bm,),
      in_specs=[pl.BlockSpec((bm, k), lambda i: (i, 0)),
                pl.BlockSpec((k, n), lambda i: (0, 0))],
      out_specs=pl.BlockSpec((bm, n), lambda i: (i, 0)),
      out_shape=jax.ShapeDtypeStruct((m, n), jnp.float32),
  )(x, w)


def _tc_mid(p, b1):
  _, n, d = p.shape

  def body(p_ref, b_ref, o_ref):
    o_ref[...] = jnp.maximum(p_ref[0] + p_ref[1] + b_ref[...], 0.0)

  return pl.pallas_call(
      body,
      out_shape=jax.ShapeDtypeStruct((n, d), jnp.float32),
  )(p, b1)


def _tc_out(q, w2, b2, n_out):
  dout = w2.shape[1]

  def body(q_ref, w_ref, b_ref, o_ref):
    acc = q_ref[0, :n_out, :] + q_ref[1, :n_out, :]
    o_ref[...] = jnp.dot(acc, w_ref[...],
                         preferred_element_type=jnp.float32) + b_ref[...]

  return pl.pallas_call(
      body,
      out_shape=jax.ShapeDtypeStruct((n_out, dout), jnp.float32),
  )(q, w2, b2)


def kernel(adj_indices, adj_values, features, W1, b1, W2, b2):
  dst = adj_indices[0].astype(jnp.int32)
  src = adj_indices[1].astype(jnp.int32)
  n = features.shape[0]
  e = dst.shape[0]
  gpw = -(-(e // GROUP) // (NW * 8)) * 8
  e_pad = NW * gpw * GROUP
  pad = e_pad - e
  src2 = jnp.concatenate([src, jnp.zeros((pad,), jnp.int32)]).reshape(-1, GROUP)
  dst2 = jnp.concatenate([dst, jnp.zeros((pad,), jnp.int32)]).reshape(-1, GROUP)
  vals2 = jnp.concatenate(
      [adj_values, jnp.zeros((pad,), jnp.float32)]).reshape(-1, GROUP)
  n_pad = -(-n // (NS * 8)) * NS * 8

  s1 = _tc_linear1(features, W1)
  p = _sc_spmm(s1, src2, dst2, vals2, n_pad).reshape(NC, n_pad, LANES)
  h = _tc_mid(p, b1.reshape(1, LANES))
  q = _sc_spmm(h, src2, dst2, vals2, n_pad).reshape(NC, n_pad, LANES)
  return _tc_out(q, W2, b2.reshape(1, -1), n)
